# initial kernel scaffold (unmeasured)
import jax
import jax.numpy as jnp
from jax import lax
from jax.experimental import pallas as pl
from jax.experimental.pallas import tpu as pltpu

N_DEV = 4


def kernel(x, w_mat, scale_x, scale_w):
    m_global, k_shard = x.shape
    _, n = w_mat.shape
    m_per = m_global // N_DEV

    def body(x_ref, w_ref, sx_ref, sw_ref, out_ref, comm_ref, send_sems, recv_sems):
        my = lax.axis_index("i")
        left = (my + N_DEV - 1) % N_DEV
        right = (my + 1) % N_DEV

        barrier_sem = pltpu.get_barrier_semaphore()
        for nbr in (left, right):
            pl.semaphore_signal(
                barrier_sem, inc=1,
                device_id=(nbr,), device_id_type=pl.DeviceIdType.MESH,
            )
        pl.semaphore_wait(barrier_sem, 2)

        def partial_chunk(c):
            xs = x_ref[pl.ds(c * m_per, m_per), :]
            return lax.dot_general(
                xs, w_ref[...],
                (((1,), (0,)), ((), ())),
                preferred_element_type=jnp.int32,
            )

        comm_ref[0] = partial_chunk((my + N_DEV - 1) % N_DEV)

        for h in range(N_DEV - 1):
            rdma = pltpu.make_async_remote_copy(
                src_ref=comm_ref.at[h],
                dst_ref=comm_ref.at[h + 1],
                send_sem=send_sems.at[h],
                recv_sem=recv_sems.at[h],
                device_id=(right,),
                device_id_type=pl.DeviceIdType.MESH,
            )
            rdma.start()
            t = partial_chunk((my + N_DEV - 2 - h) % N_DEV)
            rdma.wait()
            comm_ref[h + 1] = comm_ref[h + 1] + t

        s = sx_ref[0] * sw_ref[0]
        acc = comm_ref[N_DEV - 1].astype(jnp.float32) * s
        out_ref[...] = jnp.maximum(acc, 0.0)

    return pl.pallas_call(
        body,
        out_shape=jax.ShapeDtypeStruct((m_per, n), jnp.float32),
        in_specs=[
            pl.BlockSpec(memory_space=pltpu.VMEM),
            pl.BlockSpec(memory_space=pltpu.VMEM),
            pl.BlockSpec(memory_space=pltpu.SMEM),
            pl.BlockSpec(memory_space=pltpu.SMEM),
        ],
        out_specs=pl.BlockSpec(memory_space=pltpu.VMEM),
        scratch_shapes=[
            pltpu.VMEM((N_DEV, m_per, n), jnp.int32),
            pltpu.SemaphoreType.DMA((N_DEV - 1,)),
            pltpu.SemaphoreType.DMA((N_DEV - 1,)),
        ],
        compiler_params=pltpu.CompilerParams(collective_id=0),
    )(x, w_mat, scale_x, scale_w)


# baseline (device time: 298714 ns/iter reference)
import jax
import jax.numpy as jnp
from jax import lax
from jax.experimental import pallas as pl
from jax.experimental.pallas import tpu as pltpu

N_DEV = 4


def kernel(x, w_mat, scale_x, scale_w):
    m_global, k_shard = x.shape
    _, n = w_mat.shape
    m_per = m_global // N_DEV

    def body(x_ref, w_ref, sx_ref, sw_ref, out_ref, comm_ref, send_sems, recv_sems):
        my = lax.axis_index("i")
        left = (my + N_DEV - 1) % N_DEV
        right = (my + 1) % N_DEV

        barrier_sem = pltpu.get_barrier_semaphore()
        for nbr in (left, right):
            pl.semaphore_signal(
                barrier_sem, inc=1,
                device_id=(nbr,), device_id_type=pl.DeviceIdType.MESH,
            )
        pl.semaphore_wait(barrier_sem, 2)

        def partial_chunk(c):
            xs = x_ref[pl.ds(c * m_per, m_per), :]
            return lax.dot_general(
                xs, w_ref[...],
                (((1,), (0,)), ((), ())),
                preferred_element_type=jnp.int32,
            )

        comm_ref[0] = partial_chunk((my + N_DEV - 1) % N_DEV)

        for h in range(N_DEV - 1):
            rdma = pltpu.make_async_remote_copy(
                src_ref=comm_ref.at[h],
                dst_ref=comm_ref.at[h + 1],
                send_sem=send_sems.at[h],
                recv_sem=recv_sems.at[h],
                device_id=(right,),
                device_id_type=pl.DeviceIdType.MESH,
            )
            rdma.start()
            t = partial_chunk((my + N_DEV - 2 - h) % N_DEV)
            rdma.wait()
            comm_ref[h + 1] = comm_ref[h + 1] + t

        s = sx_ref[0] * sw_ref[0]
        acc = comm_ref[N_DEV - 1].astype(jnp.float32) * s
        out_ref[...] = jnp.maximum(acc, 0.0)

    return pl.pallas_call(
        body,
        out_shape=jax.ShapeDtypeStruct((m_per, n), jnp.float32),
        in_specs=[
            pl.BlockSpec(memory_space=pltpu.VMEM),
            pl.BlockSpec(memory_space=pltpu.VMEM),
            pl.BlockSpec(memory_space=pltpu.SMEM),
            pl.BlockSpec(memory_space=pltpu.SMEM),
        ],
        out_specs=pl.BlockSpec(memory_space=pltpu.VMEM),
        scratch_shapes=[
            pltpu.VMEM((N_DEV, m_per, n), jnp.int32),
            pltpu.SemaphoreType.DMA((N_DEV - 1,)),
            pltpu.SemaphoreType.DMA((N_DEV - 1,)),
        ],
        compiler_params=pltpu.CompilerParams(
            collective_id=0, vmem_limit_bytes=100 * 1024 * 1024
        ),
    )(x, w_mat, scale_x, scale_w)


# device time: 163828 ns/iter; 1.8233x vs baseline; 1.8233x over previous
import jax
import jax.numpy as jnp
from jax import lax
from jax.experimental import pallas as pl
from jax.experimental.pallas import tpu as pltpu

N_DEV = 4


def kernel(x, w_mat, scale_x, scale_w):
    m_global, k_shard = x.shape
    _, n = w_mat.shape
    m_per = m_global // N_DEV
    n2 = n // 2

    def body(x_ref, w_ref, sx_ref, sw_ref, out_ref,
             cw_ref, ccw_ref, cw_send, cw_recv, ccw_send, ccw_recv):
        my = lax.axis_index("i")
        left = (my + N_DEV - 1) % N_DEV
        right = (my + 1) % N_DEV

        barrier_sem = pltpu.get_barrier_semaphore()
        for nbr in (left, right):
            pl.semaphore_signal(
                barrier_sem, inc=1,
                device_id=(nbr,), device_id_type=pl.DeviceIdType.MESH,
            )
        pl.semaphore_wait(barrier_sem, 2)

        def partial_cw(c):
            xs = x_ref[pl.ds(c * m_per, m_per), :]
            return lax.dot_general(
                xs, w_ref[:, 0:n2],
                (((1,), (0,)), ((), ())),
                preferred_element_type=jnp.int32,
            )

        def partial_ccw(c):
            xs = x_ref[pl.ds(c * m_per, m_per), :]
            return lax.dot_general(
                xs, w_ref[:, n2:n],
                (((1,), (0,)), ((), ())),
                preferred_element_type=jnp.int32,
            )

        cw_ref[0] = partial_cw((my + N_DEV - 1) % N_DEV)
        ccw_ref[0] = partial_ccw((my + 1) % N_DEV)

        for h in range(N_DEV - 1):
            rdma_cw = pltpu.make_async_remote_copy(
                src_ref=cw_ref.at[h],
                dst_ref=cw_ref.at[h + 1],
                send_sem=cw_send.at[h],
                recv_sem=cw_recv.at[h],
                device_id=(right,),
                device_id_type=pl.DeviceIdType.MESH,
            )
            rdma_ccw = pltpu.make_async_remote_copy(
                src_ref=ccw_ref.at[h],
                dst_ref=ccw_ref.at[h + 1],
                send_sem=ccw_send.at[h],
                recv_sem=ccw_recv.at[h],
                device_id=(left,),
                device_id_type=pl.DeviceIdType.MESH,
            )
            rdma_cw.start()
            rdma_ccw.start()
            t_cw = partial_cw((my + N_DEV - 2 - h) % N_DEV)
            t_ccw = partial_ccw((my + 2 + h) % N_DEV)
            rdma_cw.wait()
            cw_ref[h + 1] = cw_ref[h + 1] + t_cw
            rdma_ccw.wait()
            ccw_ref[h + 1] = ccw_ref[h + 1] + t_ccw

        s = sx_ref[0] * sw_ref[0]
        out_ref[:, 0:n2] = jnp.maximum(
            cw_ref[N_DEV - 1].astype(jnp.float32) * s, 0.0)
        out_ref[:, n2:n] = jnp.maximum(
            ccw_ref[N_DEV - 1].astype(jnp.float32) * s, 0.0)

    return pl.pallas_call(
        body,
        out_shape=jax.ShapeDtypeStruct((m_per, n), jnp.float32),
        in_specs=[
            pl.BlockSpec(memory_space=pltpu.VMEM),
            pl.BlockSpec(memory_space=pltpu.VMEM),
            pl.BlockSpec(memory_space=pltpu.SMEM),
            pl.BlockSpec(memory_space=pltpu.SMEM),
        ],
        out_specs=pl.BlockSpec(memory_space=pltpu.VMEM),
        scratch_shapes=[
            pltpu.VMEM((N_DEV, m_per, n2), jnp.int32),
            pltpu.VMEM((N_DEV, m_per, n2), jnp.int32),
            pltpu.SemaphoreType.DMA((N_DEV - 1,)),
            pltpu.SemaphoreType.DMA((N_DEV - 1,)),
            pltpu.SemaphoreType.DMA((N_DEV - 1,)),
            pltpu.SemaphoreType.DMA((N_DEV - 1,)),
        ],
        compiler_params=pltpu.CompilerParams(
            collective_id=0, vmem_limit_bytes=100 * 1024 * 1024
        ),
    )(x, w_mat, scale_x, scale_w)


# device time: 155341 ns/iter; 1.9230x vs baseline; 1.0546x over previous
import jax
import jax.numpy as jnp
from jax import lax
from jax.experimental import pallas as pl
from jax.experimental.pallas import tpu as pltpu

N_DEV = 4
N_FLOW = 2


def kernel(x, w_mat, scale_x, scale_w):
    m_global, k_shard = x.shape
    _, n = w_mat.shape
    m_per = m_global // N_DEV
    m_sub = m_per // N_FLOW
    n2 = n // 2

    def body(x_ref, w_ref, sx_ref, sw_ref, out_ref,
             cw_ref, ccw_ref, cw_send, cw_recv, ccw_send, ccw_recv):
        my = lax.axis_index("i")
        left = (my + N_DEV - 1) % N_DEV
        right = (my + 1) % N_DEV

        barrier_sem = pltpu.get_barrier_semaphore()
        for nbr in (left, right):
            pl.semaphore_signal(
                barrier_sem, inc=1,
                device_id=(nbr,), device_id_type=pl.DeviceIdType.MESH,
            )
        pl.semaphore_wait(barrier_sem, 2)

        def partial(c, f, d):
            xs = x_ref[pl.ds(c * m_per + f * m_sub, m_sub), :]
            wcols = w_ref[:, 0:n2] if d == 0 else w_ref[:, n2:n]
            return lax.dot_general(
                xs, wcols,
                (((1,), (0,)), ((), ())),
                preferred_element_type=jnp.int32,
            )

        def arriving_chunk(d, h):
            return (my + N_DEV - 2 - h) % N_DEV if d == 0 else (my + 2 + h) % N_DEV

        def make(d, f, h):
            ref = cw_ref if d == 0 else ccw_ref
            ssem = cw_send if d == 0 else ccw_send
            rsem = cw_recv if d == 0 else ccw_recv
            tgt = right if d == 0 else left
            return pltpu.make_async_remote_copy(
                src_ref=ref.at[h, f],
                dst_ref=ref.at[h + 1, f],
                send_sem=ssem.at[f, h],
                recv_sem=rsem.at[f, h],
                device_id=(tgt,),
                device_id_type=pl.DeviceIdType.MESH,
            )

        ops = {}
        for f in range(N_FLOW):
            for d in range(2):
                ref = cw_ref if d == 0 else ccw_ref
                c0 = (my + N_DEV - 1) % N_DEV if d == 0 else (my + 1) % N_DEV
                ref[0, f] = partial(c0, f, d)
                ops[d, f] = make(d, f, 0)
                ops[d, f].start()

        for h in range(N_DEV - 1):
            t = {(d, f): partial(arriving_chunk(d, h), f, d)
                 for f in range(N_FLOW) for d in range(2)}
            for f in range(N_FLOW):
                for d in range(2):
                    ref = cw_ref if d == 0 else ccw_ref
                    ops[d, f].wait()
                    acc = ref[h + 1, f] + t[d, f]
                    if h < N_DEV - 2:
                        ref[h + 1, f] = acc
                        ops[d, f] = make(d, f, h + 1)
                        ops[d, f].start()
                    else:
                        s = sx_ref[0] * sw_ref[0]
                        y = jnp.maximum(acc.astype(jnp.float32) * s, 0.0)
                        rows = pl.ds(f * m_sub, m_sub)
                        if d == 0:
                            out_ref[rows, 0:n2] = y
                        else:
                            out_ref[rows, n2:n] = y

    return pl.pallas_call(
        body,
        out_shape=jax.ShapeDtypeStruct((m_per, n), jnp.float32),
        in_specs=[
            pl.BlockSpec(memory_space=pltpu.VMEM),
            pl.BlockSpec(memory_space=pltpu.VMEM),
            pl.BlockSpec(memory_space=pltpu.SMEM),
            pl.BlockSpec(memory_space=pltpu.SMEM),
        ],
        out_specs=pl.BlockSpec(memory_space=pltpu.VMEM),
        scratch_shapes=[
            pltpu.VMEM((N_DEV, N_FLOW, m_sub, n2), jnp.int32),
            pltpu.VMEM((N_DEV, N_FLOW, m_sub, n2), jnp.int32),
            pltpu.SemaphoreType.DMA((N_FLOW, N_DEV - 1)),
            pltpu.SemaphoreType.DMA((N_FLOW, N_DEV - 1)),
            pltpu.SemaphoreType.DMA((N_FLOW, N_DEV - 1)),
            pltpu.SemaphoreType.DMA((N_FLOW, N_DEV - 1)),
        ],
        compiler_params=pltpu.CompilerParams(
            collective_id=0, vmem_limit_bytes=100 * 1024 * 1024
        ),
    )(x, w_mat, scale_x, scale_w)
